# baseline (speedup 1.0000x reference)
import jax
import jax.numpy as jnp
from jax.experimental import pallas as pl
from jax.experimental.pallas import tpu as pltpu

_R = 512
_C = 128
_N = 4096


def _body(x_ref, u_ref, o_ref):
    u = u_ref[...]
    carry = jnp.zeros((_R, 1), jnp.float32)
    for c in range(_N // _C):
        blk = x_ref[:, c * _C:(c + 1) * _C]
        y = jax.lax.dot_general(
            blk, u, (((1,), (0,)), ((), ())),
            preferred_element_type=jnp.float32,
            precision=jax.lax.Precision.DEFAULT,
        )
        y = y + carry
        o_ref[:, c * _C:(c + 1) * _C] = y
        carry = y[:, _C - 1:_C]


def kernel(x):
    orig_dtype = x.dtype
    xf = x.astype(jnp.float32)
    B, S, N = xf.shape
    M = B * S
    x2 = xf.reshape(M, N)
    U = jnp.triu(jnp.ones((_C, _C), jnp.float32))
    grid = (M // _R,)
    out = pl.pallas_call(
        _body,
        grid=grid,
        in_specs=[
            pl.BlockSpec((_R, N), lambda i: (i, 0)),
            pl.BlockSpec((_C, _C), lambda i: (0, 0)),
        ],
        out_specs=pl.BlockSpec((_R, N), lambda i: (i, 0)),
        out_shape=jax.ShapeDtypeStruct((M, N), jnp.float32),
        compiler_params=pltpu.CompilerParams(
            dimension_semantics=("parallel",),
        ),
    )(x2, U)
    return out.reshape(B, S, N).astype(orig_dtype)


# speedup vs baseline: 1.0008x; 1.0008x over previous
import jax
import jax.numpy as jnp
from jax.experimental import pallas as pl
from jax.experimental.pallas import tpu as pltpu

_R = 1024
_W = 2048
_C = 128
_N = 4096


def _body(x_ref, u_ref, o_ref, carry_ref):
    j = pl.program_id(1)
    u = u_ref[...]

    @pl.when(j == 0)
    def _():
        carry_ref[...] = jnp.zeros((_R, 1), jnp.float32)

    carry = carry_ref[...]
    for c in range(_W // _C):
        blk = x_ref[:, c * _C:(c + 1) * _C]
        y = jax.lax.dot_general(
            blk, u, (((1,), (0,)), ((), ())),
            preferred_element_type=jnp.float32,
            precision=jax.lax.Precision.DEFAULT,
        )
        y = y + carry
        o_ref[:, c * _C:(c + 1) * _C] = y
        carry = y[:, _C - 1:_C]
    carry_ref[...] = carry


def kernel(x):
    orig_dtype = x.dtype
    xf = x.astype(jnp.float32)
    B, S, N = xf.shape
    M = B * S
    x2 = xf.reshape(M, N)
    U = jnp.triu(jnp.ones((_C, _C), jnp.float32))
    grid = (M // _R, N // _W)
    out = pl.pallas_call(
        _body,
        grid=grid,
        in_specs=[
            pl.BlockSpec((_R, _W), lambda i, j: (i, j)),
            pl.BlockSpec((_C, _C), lambda i, j: (0, 0)),
        ],
        out_specs=pl.BlockSpec((_R, _W), lambda i, j: (i, j)),
        out_shape=jax.ShapeDtypeStruct((M, N), jnp.float32),
        scratch_shapes=[pltpu.VMEM((_R, 1), jnp.float32)],
        compiler_params=pltpu.CompilerParams(
            dimension_semantics=("parallel", "arbitrary"),
        ),
    )(x2, U)
    return out.reshape(B, S, N).astype(orig_dtype)
